# Initial kernel scaffold; baseline (speedup 1.0000x reference)
import jax
import jax.numpy as jnp
from jax.experimental import pallas as pl


def kernel(x, edge_index, edge_weight, filter_param, chebynodes_vals):
    raise NotImplementedError("write your pallas kernel here")


# speedup vs baseline: 4.3057x; 4.3057x over previous
import functools

import jax
import jax.numpy as jnp
from jax import lax
from jax.experimental import pallas as pl
from jax.experimental.pallas import tpu as pltpu
from jax.experimental.pallas import tpu_sc as plsc

K = 8
N = 10000
E = 320000
D = 128

NC = 2
NS = 16
L = 16

H = D // NC
ET = E // NS
CB = 128
NCH = (ET + CB - 1) // CB
ETP = NCH * CB

NP = 10240
RPT = NP // NS
RCH = 64
NRC = RPT // RCH


def _sc_body(xg, colp, rowp, wp, coef, yout,
             g1, g2, c_sp,
             col_v, row_v, w_v, rbuf, ridx, cbuf, gbuf, ybuf, coef_v,
             gsem, ssem):
  c = lax.axis_index("c")
  s = lax.axis_index("s")

  pltpu.sync_copy(colp.at[s], col_v)
  pltpu.sync_copy(rowp.at[s], row_v)
  pltpu.sync_copy(wp.at[s], w_v)
  pltpu.sync_copy(coef, coef_v)

  cN = jnp.full((L,), c * NP, dtype=jnp.int32)

  def adj_body(t, _):
    i = t // (CB // L)
    k = t % (CB // L)
    sl = pl.ds(k * L, L)
    col_v[i, sl] = col_v[i, sl] + cN
    return 0
  lax.fori_loop(0, NCH * (CB // L), adj_body, 0)

  z16 = jnp.zeros((L,), jnp.float32)

  def zero_rbuf():
    def zero_body(t, _):
      r = t // (H // L)
      k = t % (H // L)
      rbuf[r, pl.ds(k * L, L)] = z16
      return 0
    lax.fori_loop(0, CB * (H // L), zero_body, 0)

  zero_rbuf()
  for ch in range(RPT // CB):
    pltpu.sync_copy(rbuf, c_sp.at[pl.ds(s * RPT + ch * CB, CB)])
  plsc.subcore_barrier()

  def edge_phase(src):
    def chunk_body(j, _):
      for k in range(CB // L):
        sl = pl.ds(k * L, L)
        ridx[sl] = row_v[j, sl]
      pltpu.async_copy(src.at[col_v.at[j]], rbuf, gsem).wait()
      jb = j * CB

      def gbody(g, _):
        wg = w_v[pl.ds(jb + g * L, L)]
        for el in range(L):
          wv = jnp.full((L,), wg[el], dtype=jnp.float32)
          e = g * L + el
          for f in range(H // L):
            sl = pl.ds(f * L, L)
            rbuf[e, sl] = rbuf[e, sl] * wv
        return 0
      lax.fori_loop(0, CB // L, gbody, 0)
      pltpu.async_copy(rbuf, c_sp.at[ridx], ssem, add=True).wait()
      return 0
    lax.fori_loop(0, NCH, chunk_body, 0)

  def linear_phase(i, prev, dest):
    cvec = coef_v[:]
    ci = jnp.full((L,), cvec[i], dtype=jnp.float32)
    c0 = jnp.full((L,), cvec[0], dtype=jnp.float32)
    zero_rbuf()
    for ch in range(NRC):
      r0 = s * RPT + ch * RCH
      g0 = c * NP + r0
      pltpu.sync_copy(c_sp.at[pl.ds(r0, RCH)], cbuf)
      pltpu.sync_copy(rbuf.at[pl.ds(0, RCH)], c_sp.at[pl.ds(r0, RCH)])
      pltpu.sync_copy(prev.at[pl.ds(g0, RCH)], gbuf)
      if i > 1:
        pltpu.sync_copy(yout.at[c, pl.ds(r0, RCH)], ybuf)

      def vbody(t, _):
        r = t // (H // L)
        k = t % (H // L)
        sl = pl.ds(k * L, L)
        cv = cbuf[r, sl]
        gv = gbuf[r, sl]
        if i == 1:
          ybuf[r, sl] = c0 * gv + ci * cv
        else:
          tv = 2.0 * cv - gv
          cbuf[r, sl] = tv
          ybuf[r, sl] = ybuf[r, sl] + ci * tv
        return 0
      lax.fori_loop(0, RCH * (H // L), vbody, 0)

      pltpu.sync_copy(cbuf, dest.at[pl.ds(g0, RCH)])
      pltpu.sync_copy(ybuf, yout.at[c, pl.ds(r0, RCH)])

  src = xg
  for i in range(1, K + 1):
    edge_phase(src)
    plsc.subcore_barrier()
    if i == 1:
      prev, dest = xg, g1
    elif i == 2:
      prev, dest = xg, g2
    else:
      dest = g1 if (i % 2 == 1) else g2
      prev = dest
    linear_phase(i, prev, dest)
    plsc.subcore_barrier()
    src = dest


@jax.jit
def _cheb_sc(xg, colp, rowp, wp, coef):
  mesh = plsc.VectorSubcoreMesh(core_axis_name="c", subcore_axis_name="s")
  f = pl.kernel(
      _sc_body,
      out_type=jax.ShapeDtypeStruct((NC, NP, H), jnp.float32),
      mesh=mesh,
      compiler_params=pltpu.CompilerParams(use_tc_tiling_on_sc=False),
      scratch_types=[
          pltpu.HBM((NC * NP, H), jnp.float32),
          pltpu.HBM((NC * NP, H), jnp.float32),
          pltpu.VMEM_SHARED((NP, H), jnp.float32),
          pltpu.VMEM((NCH, CB), jnp.int32),
          pltpu.VMEM((NCH, CB), jnp.int32),
          pltpu.VMEM((ETP,), jnp.float32),
          pltpu.VMEM((CB, H), jnp.float32),
          pltpu.VMEM((CB,), jnp.int32),
          pltpu.VMEM((RCH, H), jnp.float32),
          pltpu.VMEM((RCH, H), jnp.float32),
          pltpu.VMEM((RCH, H), jnp.float32),
          pltpu.VMEM((L,), jnp.float32),
          pltpu.SemaphoreType.DMA,
          pltpu.SemaphoreType.DMA,
      ],
  )
  return f(xg, colp, rowp, wp, coef)


def kernel(x, edge_index, edge_weight, filter_param, chebynodes_vals):
  fp = jax.nn.relu(filter_param)
  fp = chebynodes_vals @ fp
  fp = 2.0 * fp / (K + 1)
  fp = fp.at[0].set(fp[0] / 2.0)
  coef = jnp.zeros((L,), jnp.float32).at[:K + 1].set(fp[:, 0])

  row = edge_index[0].reshape(NS, ET)
  col = edge_index[1].reshape(NS, ET)
  w = (-edge_weight).reshape(NS, ET)
  pad = ((0, 0), (0, ETP - ET))
  rowp = jnp.pad(row, pad).reshape(NS, NCH, CB)
  colp = jnp.pad(col, pad).reshape(NS, NCH, CB)
  wp = jnp.pad(w, pad)

  xp = jnp.pad(x, ((0, NP - N), (0, 0)))
  xg = xp.reshape(NP, NC, H).transpose(1, 0, 2).reshape(NC * NP, H)

  yout = _cheb_sc(xg, colp, rowp, wp, coef)
  return jnp.concatenate([yout[0, :N], yout[1, :N]], axis=1)


# speedup vs baseline: 5.0605x; 1.1753x over previous
import jax
import jax.numpy as jnp
from jax import lax
from jax.experimental import pallas as pl
from jax.experimental.pallas import tpu as pltpu
from jax.experimental.pallas import tpu_sc as plsc

K = 8
N = 10000
E = 320000
D = 128

NC = 2
NS = 16
L = 16

H = D // NC
ET = E // NS
CB = 128
NCH = 159
ETP = NCH * CB

NP = 10240
RPT = NP // NS
RCH = 64
NRC = RPT // RCH


def _sc_body(xg, crowp, wp, coef, yout,
             g1, g2, c_sp,
             crow_v, w_v, rb0, rb1, rb2, ri0, ri1, ri2, ci0, ci1, ci2,
             cbuf, gbuf, ybuf, coef_v,
             gs0, gs1, gs2, ss0, ss1, ss2):
  c = lax.axis_index("c")
  s = lax.axis_index("s")

  rbuf = (rb0, rb1, rb2)
  ridx = (ri0, ri1, ri2)
  cidx = (ci0, ci1, ci2)
  gsem = (gs0, gs1, gs2)
  ssem = (ss0, ss1, ss2)

  pltpu.sync_copy(crowp.at[s], crow_v)
  pltpu.sync_copy(wp.at[s], w_v)
  pltpu.sync_copy(coef, coef_v)

  cN = jnp.full((L,), c * NP, dtype=jnp.int32)
  m16 = jnp.full((L,), 0xFFFF, dtype=jnp.int32)
  z16 = jnp.zeros((L,), jnp.float32)

  def zero_rows(nrows):
    def zb(t, _):
      r = t // (H // L)
      k = t % (H // L)
      rb0[r, pl.ds(k * L, L)] = z16
      return 0
    lax.fori_loop(0, nrows * (H // L), zb, 0)

  zero_rows(CB)
  for ch in range(RPT // CB):
    pltpu.sync_copy(rb0, c_sp.at[pl.ds(s * RPT + ch * CB, CB)])
  for ch in range(NRC):
    g0 = c * NP + s * RPT + ch * RCH
    pltpu.sync_copy(xg.at[pl.ds(g0, RCH)], gbuf)
    pltpu.sync_copy(gbuf, g1.at[pl.ds(g0, RCH)])
  plsc.subcore_barrier()

  def stage_and_gather(j, q, src):
    for k in range(CB // L):
      sl = pl.ds(k * L, L)
      v = crow_v[j, sl]
      ridx[q][sl] = v & m16
      cidx[q][sl] = (v >> 16) + cN
    pltpu.async_copy(src.at[cidx[q]], rbuf[q], gsem[q])

  def wait_gather(q, src):
    pltpu.make_async_copy(src.at[pl.ds(0, CB)], rbuf[q], gsem[q]).wait()

  def wait_scatter(q):
    pltpu.make_async_copy(rbuf[q], c_sp.at[pl.ds(0, CB)], ssem[q]).wait()

  def scale(j, q):
    jb = j * CB
    b = rbuf[q]

    def gbody(g, _):
      wg = w_v[pl.ds(jb + g * L, L)]
      for el in range(L):
        wv = jnp.full((L,), wg[el], dtype=jnp.float32)
        e = g * L + el
        for f in range(H // L):
          sl = pl.ds(f * L, L)
          b[e, sl] = b[e, sl] * wv
      return 0
    lax.fori_loop(0, CB // L, gbody, 0)

  def scatter(j, q):
    pltpu.async_copy(rbuf[q], c_sp.at[ridx[q]], ssem[q], add=True)

  def slot(j, q, src, prep, swait):
    wait_gather(q, src)
    qn = (q + 2) % 3
    if prep is not None:
      def _prep():
        if swait:
          wait_scatter(qn)
        stage_and_gather(j + 2, qn, src)
      if prep is True:
        _prep()
      else:
        pl.when(prep)(_prep)
    scale(j, q)
    scatter(j, q)

  def edge_phase(src):
    stage_and_gather(0, 0, src)
    stage_and_gather(1, 1, src)

    def tri_body(p, _):
      j0 = 3 * p
      wait_gather(0, src)
      def _prep0():
        def _sw():
          wait_scatter(2)
        pl.when(p > 0)(_sw)
        stage_and_gather(j0 + 2, 2, src)
      _prep0()
      scale(j0, 0)
      scatter(j0, 0)
      slot(j0 + 1, 1, src, prep=(p < (NCH - 3) // 3), swait=True)
      slot(j0 + 2, 2, src, prep=(p < (NCH - 3) // 3), swait=True)
      return 0
    lax.fori_loop(0, NCH // 3, tri_body, 0)

    wait_scatter(0)
    wait_scatter(1)
    wait_scatter(2)

  def linear_1():
    cvec = coef_v[:]
    c0 = jnp.full((L,), cvec[0], dtype=jnp.float32)
    c1 = jnp.full((L,), cvec[1], dtype=jnp.float32)
    zero_rows(RCH)
    for ch in range(NRC):
      r0 = s * RPT + ch * RCH
      g0 = c * NP + r0
      pltpu.sync_copy(c_sp.at[pl.ds(r0, RCH)], cbuf)
      pltpu.sync_copy(rb0.at[pl.ds(0, RCH)], c_sp.at[pl.ds(r0, RCH)])
      pltpu.sync_copy(g1.at[pl.ds(g0, RCH)], gbuf)

      def vbody(t, _):
        r = t // (H // L)
        k = t % (H // L)
        sl = pl.ds(k * L, L)
        ybuf[r, sl] = c0 * gbuf[r, sl] + c1 * cbuf[r, sl]
        return 0
      lax.fori_loop(0, RCH * (H // L), vbody, 0)

      pltpu.sync_copy(gbuf, g2.at[pl.ds(g0, RCH)])
      pltpu.sync_copy(cbuf, g1.at[pl.ds(g0, RCH)])
      pltpu.sync_copy(ybuf, yout.at[c, pl.ds(r0, RCH)])

  def linear_dyn(i):
    civ = coef_v[pl.ds(i, L)]
    ci = jnp.full((L,), civ[0], dtype=jnp.float32)
    zero_rows(RCH)
    hb = rb1.at[pl.ds(0, RCH)]
    for ch in range(NRC):
      r0 = s * RPT + ch * RCH
      g0 = c * NP + r0
      pltpu.sync_copy(c_sp.at[pl.ds(r0, RCH)], cbuf)
      pltpu.sync_copy(rb0.at[pl.ds(0, RCH)], c_sp.at[pl.ds(r0, RCH)])
      pltpu.sync_copy(g2.at[pl.ds(g0, RCH)], gbuf)
      pltpu.sync_copy(g1.at[pl.ds(g0, RCH)], hb)
      pltpu.sync_copy(hb, g2.at[pl.ds(g0, RCH)])
      pltpu.sync_copy(yout.at[c, pl.ds(r0, RCH)], ybuf)

      def vbody(t, _):
        r = t // (H // L)
        k = t % (H // L)
        sl = pl.ds(k * L, L)
        tv = 2.0 * cbuf[r, sl] - gbuf[r, sl]
        cbuf[r, sl] = tv
        ybuf[r, sl] = ybuf[r, sl] + ci * tv
        return 0
      lax.fori_loop(0, RCH * (H // L), vbody, 0)

      pltpu.sync_copy(cbuf, g1.at[pl.ds(g0, RCH)])
      pltpu.sync_copy(ybuf, yout.at[c, pl.ds(r0, RCH)])

  edge_phase(g1)
  plsc.subcore_barrier()
  linear_1()
  plsc.subcore_barrier()

  def step_body(p, _):
    edge_phase(g1)
    plsc.subcore_barrier()
    linear_dyn(p + 2)
    plsc.subcore_barrier()
    return 0
  lax.fori_loop(0, K - 1, step_body, 0)


@jax.jit
def _cheb_sc(xg, crowp, wp, coef):
  mesh = plsc.VectorSubcoreMesh(core_axis_name="c", subcore_axis_name="s")
  f = pl.kernel(
      _sc_body,
      out_type=jax.ShapeDtypeStruct((NC, NP, H), jnp.float32),
      mesh=mesh,
      compiler_params=pltpu.CompilerParams(use_tc_tiling_on_sc=False),
      scratch_types=[
          pltpu.HBM((NC * NP, H), jnp.float32),
          pltpu.HBM((NC * NP, H), jnp.float32),
          pltpu.VMEM_SHARED((NP, H), jnp.float32),
          pltpu.VMEM((NCH, CB), jnp.int32),
          pltpu.VMEM((ETP,), jnp.float32),
          pltpu.VMEM((CB, H), jnp.float32),
          pltpu.VMEM((CB, H), jnp.float32),
          pltpu.VMEM((CB, H), jnp.float32),
          pltpu.VMEM((CB,), jnp.int32),
          pltpu.VMEM((CB,), jnp.int32),
          pltpu.VMEM((CB,), jnp.int32),
          pltpu.VMEM((CB,), jnp.int32),
          pltpu.VMEM((CB,), jnp.int32),
          pltpu.VMEM((CB,), jnp.int32),
          pltpu.VMEM((RCH, H), jnp.float32),
          pltpu.VMEM((RCH, H), jnp.float32),
          pltpu.VMEM((RCH, H), jnp.float32),
          pltpu.VMEM((2 * L,), jnp.float32),
          pltpu.SemaphoreType.DMA,
          pltpu.SemaphoreType.DMA,
          pltpu.SemaphoreType.DMA,
          pltpu.SemaphoreType.DMA,
          pltpu.SemaphoreType.DMA,
          pltpu.SemaphoreType.DMA,
      ],
  )
  return f(xg, crowp, wp, coef)


def kernel(x, edge_index, edge_weight, filter_param, chebynodes_vals):
  fp = jax.nn.relu(filter_param)
  fp = chebynodes_vals @ fp
  fp = 2.0 * fp / (K + 1)
  fp = fp.at[0].set(fp[0] / 2.0)
  coef = jnp.zeros((2 * L,), jnp.float32).at[:K + 1].set(fp[:, 0])

  row = edge_index[0].reshape(NS, ET)
  col = edge_index[1].reshape(NS, ET)
  w = (-edge_weight).reshape(NS, ET)
  pad = ((0, 0), (0, ETP - ET))
  crowp = jnp.pad((col << 16) | row, pad).reshape(NS, NCH, CB)
  wp = jnp.pad(w, pad)

  xp = jnp.pad(x, ((0, NP - N), (0, 0)))
  xg = xp.reshape(NP, NC, H).transpose(1, 0, 2).reshape(NC * NP, H)

  yout = _cheb_sc(xg, crowp, wp, coef)
  return jnp.concatenate([yout[0, :N], yout[1, :N]], axis=1)


# speedup vs baseline: 6.3638x; 1.2575x over previous
import jax
import jax.numpy as jnp
from jax import lax
from jax.experimental import pallas as pl
from jax.experimental.pallas import tpu as pltpu
from jax.experimental.pallas import tpu_sc as plsc

K = 8
N = 10000
E = 320000
D = 128

NC = 2
NS = 16
L = 16

H = D // NC
ET = E // NS
CB = 128
NCH = 159
ETP = NCH * CB

NP = 10240
RPT = NP // NS
RCH = 128
NRC = RPT // RCH


def _sc_body(xg, crowp, wp, coef, yout,
             g1, g2, c_sp,
             crow_v, w_v, rb0, rb1, rb2, ri0, ri1, ri2, ci0, ci1, ci2,
             cbuf, coef_v,
             gs0, gs1, gs2, ss0, ss1, ss2):
  c = lax.axis_index("c")
  s = lax.axis_index("s")

  rbuf = (rb0, rb1, rb2)
  ridx = (ri0, ri1, ri2)
  cidx = (ci0, ci1, ci2)
  gsem = (gs0, gs1, gs2)
  ssem = (ss0, ss1, ss2)

  pltpu.sync_copy(crowp.at[s], crow_v)
  pltpu.sync_copy(wp.at[s], w_v)
  pltpu.sync_copy(coef, coef_v)

  cN = jnp.full((L,), c * NP, dtype=jnp.int32)
  m16 = jnp.full((L,), 0xFFFF, dtype=jnp.int32)
  z16 = jnp.zeros((L,), jnp.float32)

  def zero_rows(nrows):
    @plsc.parallel_loop(0, nrows * (H // L), step=1, unroll=2)
    def zb(t):
      r = t // (H // L)
      k = t % (H // L)
      rb0[r, pl.ds(k * L, L)] = z16

  zero_rows(CB)
  for ch in range(RPT // CB):
    pltpu.sync_copy(rb0, c_sp.at[pl.ds(s * RPT + ch * CB, CB)])
  for ch in range(NRC):
    g0 = c * NP + s * RPT + ch * RCH
    pltpu.sync_copy(xg.at[pl.ds(g0, RCH)], rb1)
    pltpu.sync_copy(rb1, g1.at[pl.ds(g0, RCH)])
  plsc.subcore_barrier()

  def stage_and_gather(j, q, src):
    for k in range(CB // L):
      sl = pl.ds(k * L, L)
      v = crow_v[j, sl]
      ridx[q][sl] = v & m16
      cidx[q][sl] = (v >> 16) + cN
    pltpu.async_copy(src.at[cidx[q]], rbuf[q], gsem[q])

  def wait_gather(q, src):
    pltpu.make_async_copy(src.at[pl.ds(0, CB)], rbuf[q], gsem[q]).wait()

  def wait_scatter(q):
    pltpu.make_async_copy(rbuf[q], c_sp.at[pl.ds(0, CB)], ssem[q]).wait()

  def scale(j, q):
    jb = j * CB
    b = rbuf[q]

    @plsc.parallel_loop(0, CB // L, step=1, unroll=2)
    def gbody(g):
      wg = w_v[pl.ds(jb + g * L, L)]
      for el in range(L):
        wv = jnp.full((L,), wg[el], dtype=jnp.float32)
        e = g * L + el
        for f in range(H // L):
          sl = pl.ds(f * L, L)
          b[e, sl] = b[e, sl] * wv

  def scatter(j, q):
    pltpu.async_copy(rbuf[q], c_sp.at[ridx[q]], ssem[q], add=True)

  def slot(j, q, src, prep, swait):
    wait_gather(q, src)
    qn = (q + 2) % 3
    if prep is not None:
      def _prep():
        if swait:
          wait_scatter(qn)
        stage_and_gather(j + 2, qn, src)
      if prep is True:
        _prep()
      else:
        pl.when(prep)(_prep)
    scale(j, q)
    scatter(j, q)

  def edge_phase(src):
    stage_and_gather(0, 0, src)
    stage_and_gather(1, 1, src)

    def tri_body(p, _):
      j0 = 3 * p
      wait_gather(0, src)
      def _prep0():
        def _sw():
          wait_scatter(2)
        pl.when(p > 0)(_sw)
        stage_and_gather(j0 + 2, 2, src)
      _prep0()
      scale(j0, 0)
      scatter(j0, 0)
      slot(j0 + 1, 1, src, prep=(p < (NCH - 3) // 3), swait=True)
      slot(j0 + 2, 2, src, prep=(p < (NCH - 3) // 3), swait=True)
      return 0
    lax.fori_loop(0, NCH // 3, tri_body, 0)

    wait_scatter(0)
    wait_scatter(1)
    wait_scatter(2)

  def linear_1():
    cvec = coef_v[:]
    c0 = jnp.full((L,), cvec[0], dtype=jnp.float32)
    c1 = jnp.full((L,), cvec[1], dtype=jnp.float32)
    zero_rows(RCH)
    for ch in range(NRC):
      r0 = s * RPT + ch * RCH
      g0 = c * NP + r0
      pltpu.sync_copy(c_sp.at[pl.ds(r0, RCH)], cbuf)
      pltpu.sync_copy(rb0, c_sp.at[pl.ds(r0, RCH)])
      pltpu.sync_copy(g1.at[pl.ds(g0, RCH)], rb1)

      @plsc.parallel_loop(0, RCH * (H // L), step=1, unroll=2)
      def vbody(t):
        r = t // (H // L)
        k = t % (H // L)
        sl = pl.ds(k * L, L)
        rb2[r, sl] = c0 * rb1[r, sl] + c1 * cbuf[r, sl]

      pltpu.sync_copy(rb1, g2.at[pl.ds(g0, RCH)])
      pltpu.sync_copy(cbuf, g1.at[pl.ds(g0, RCH)])
      pltpu.sync_copy(rb2, yout.at[c, pl.ds(r0, RCH)])

  def linear_dyn(i):
    civ = coef_v[pl.ds(i, L)]
    ci = jnp.full((L,), civ[0], dtype=jnp.float32)
    zero_rows(RCH)
    for ch in range(NRC):
      r0 = s * RPT + ch * RCH
      g0 = c * NP + r0
      pltpu.sync_copy(g2.at[pl.ds(g0, RCH)], rb1)
      pltpu.sync_copy(g1.at[pl.ds(g0, RCH)], cbuf)
      pltpu.sync_copy(cbuf, g2.at[pl.ds(g0, RCH)])
      pltpu.sync_copy(c_sp.at[pl.ds(r0, RCH)], cbuf)
      pltpu.sync_copy(rb0, c_sp.at[pl.ds(r0, RCH)])
      pltpu.sync_copy(yout.at[c, pl.ds(r0, RCH)], rb2)

      @plsc.parallel_loop(0, RCH * (H // L), step=1, unroll=2)
      def vbody(t):
        r = t // (H // L)
        k = t % (H // L)
        sl = pl.ds(k * L, L)
        tv = 2.0 * cbuf[r, sl] - rb1[r, sl]
        cbuf[r, sl] = tv
        rb2[r, sl] = rb2[r, sl] + ci * tv

      pltpu.sync_copy(cbuf, g1.at[pl.ds(g0, RCH)])
      pltpu.sync_copy(rb2, yout.at[c, pl.ds(r0, RCH)])

  edge_phase(g1)
  plsc.subcore_barrier()
  linear_1()
  plsc.subcore_barrier()

  def step_body(p, _):
    edge_phase(g1)
    plsc.subcore_barrier()
    linear_dyn(p + 2)
    plsc.subcore_barrier()
    return 0
  lax.fori_loop(0, K - 1, step_body, 0)


@jax.jit
def _cheb_sc(xg, crowp, wp, coef):
  mesh = plsc.VectorSubcoreMesh(core_axis_name="c", subcore_axis_name="s")
  f = pl.kernel(
      _sc_body,
      out_type=jax.ShapeDtypeStruct((NC, NP, H), jnp.float32),
      mesh=mesh,
      compiler_params=pltpu.CompilerParams(use_tc_tiling_on_sc=False),
      scratch_types=[
          pltpu.HBM((NC * NP, H), jnp.float32),
          pltpu.HBM((NC * NP, H), jnp.float32),
          pltpu.VMEM_SHARED((NP, H), jnp.float32),
          pltpu.VMEM((NCH, CB), jnp.int32),
          pltpu.VMEM((ETP,), jnp.float32),
          pltpu.VMEM((CB, H), jnp.float32),
          pltpu.VMEM((CB, H), jnp.float32),
          pltpu.VMEM((CB, H), jnp.float32),
          pltpu.VMEM((CB,), jnp.int32),
          pltpu.VMEM((CB,), jnp.int32),
          pltpu.VMEM((CB,), jnp.int32),
          pltpu.VMEM((CB,), jnp.int32),
          pltpu.VMEM((CB,), jnp.int32),
          pltpu.VMEM((CB,), jnp.int32),
          pltpu.VMEM((RCH, H), jnp.float32),
          pltpu.VMEM((2 * L,), jnp.float32),
          pltpu.SemaphoreType.DMA,
          pltpu.SemaphoreType.DMA,
          pltpu.SemaphoreType.DMA,
          pltpu.SemaphoreType.DMA,
          pltpu.SemaphoreType.DMA,
          pltpu.SemaphoreType.DMA,
      ],
  )
  return f(xg, crowp, wp, coef)


def kernel(x, edge_index, edge_weight, filter_param, chebynodes_vals):
  fp = jax.nn.relu(filter_param)
  fp = chebynodes_vals @ fp
  fp = 2.0 * fp / (K + 1)
  fp = fp.at[0].set(fp[0] / 2.0)
  coef = jnp.zeros((2 * L,), jnp.float32).at[:K + 1].set(fp[:, 0])

  row = edge_index[0].reshape(NS, ET)
  col = edge_index[1].reshape(NS, ET)
  w = (-edge_weight).reshape(NS, ET)
  pad = ((0, 0), (0, ETP - ET))
  crowp = jnp.pad((col << 16) | row, pad).reshape(NS, NCH, CB)
  wp = jnp.pad(w, pad)

  xp = jnp.pad(x, ((0, NP - N), (0, 0)))
  xg = xp.reshape(NP, NC, H).transpose(1, 0, 2).reshape(NC * NP, H)

  yout = _cheb_sc(xg, crowp, wp, coef)
  return jnp.concatenate([yout[0, :N], yout[1, :N]], axis=1)
